# baseline (speedup 1.0000x reference)
from opt import run_opt


def kernel(hidden_states, Wr, br, W1, b1, W2, b2, Wh, bh):
    return run_opt(hidden_states, Wr, br, W1, b1, W2, b2, Wh, bh)
